# baseline (speedup 1.0000x reference)
import jax
import jax.numpy as jnp
from jax.experimental import pallas as pl
from jax.experimental.pallas import tpu as pltpu

_TM = 1024
_NT = 8


def _fused_kernel(x_hbm, w1_hbm, w2_hbm, o_hbm,
                  xbuf, obuf, w1_ref, w2_ref, wc_ref,
                  in_sems, out_sems, w_sems):
    D = w1_ref.shape[0]
    half = D // 2

    pltpu.make_async_copy(
        w1_hbm.at[pl.ds(0, half), :], w1_ref.at[pl.ds(0, half), :],
        w_sems.at[0]).start()
    pltpu.make_async_copy(w2_hbm, w2_ref, w_sems.at[1]).start()
    pltpu.make_async_copy(
        w1_hbm.at[pl.ds(half, half), :], w1_ref.at[pl.ds(half, half), :],
        w_sems.at[2]).start()
    for i in range(_NT):
        pltpu.make_async_copy(
            x_hbm.at[pl.ds(i * _TM, _TM), :], xbuf.at[i], in_sems.at[i]
        ).start()

    pltpu.make_async_copy(
        w1_hbm.at[pl.ds(0, half), :], w1_ref.at[pl.ds(0, half), :],
        w_sems.at[0]).wait()
    pltpu.make_async_copy(w2_hbm, w2_ref, w_sems.at[1]).wait()
    wc_ref[:half, :] = jax.lax.dot_general(
        w1_ref[:half, :], w2_ref[...],
        dimension_numbers=(((1,), (1,)), ((), ())),
        preferred_element_type=jnp.float32).astype(jnp.bfloat16)
    pltpu.make_async_copy(
        w1_hbm.at[pl.ds(half, half), :], w1_ref.at[pl.ds(half, half), :],
        w_sems.at[2]).wait()
    wc_ref[half:, :] = jax.lax.dot_general(
        w1_ref[half:, :], w2_ref[...],
        dimension_numbers=(((1,), (1,)), ((), ())),
        preferred_element_type=jnp.float32).astype(jnp.bfloat16)

    for i in range(_NT):
        pltpu.make_async_copy(
            x_hbm.at[pl.ds(i * _TM, _TM), :], xbuf.at[i], in_sems.at[i]
        ).wait()
        slot = i % 2
        if i >= 2:
            pltpu.make_async_copy(
                obuf.at[slot], o_hbm.at[pl.ds((i - 2) * _TM, _TM), :],
                out_sems.at[i - 2]
            ).wait()
        obuf[slot] = jnp.dot(
            xbuf[i], wc_ref[...],
            preferred_element_type=jnp.float32).astype(jnp.bfloat16)
        pltpu.make_async_copy(
            obuf.at[slot], o_hbm.at[pl.ds(i * _TM, _TM), :], out_sems.at[i]
        ).start()

    for i in range(_NT - 2, _NT):
        pltpu.make_async_copy(
            obuf.at[i % 2], o_hbm.at[pl.ds(i * _TM, _TM), :], out_sems.at[i]
        ).wait()


def kernel(x, gate_w, bias, W1, W2):
    Bq, S, D = x.shape
    x_flat = x.reshape(-1, D)
    T = x_flat.shape[0]
    inter = W1.shape[2]
    out = pl.pallas_call(
        _fused_kernel,
        grid=(1,),
        in_specs=[
            pl.BlockSpec(memory_space=pl.ANY),
            pl.BlockSpec(memory_space=pl.ANY),
            pl.BlockSpec(memory_space=pl.ANY),
        ],
        out_specs=pl.BlockSpec(memory_space=pl.ANY),
        out_shape=jax.ShapeDtypeStruct((T, D), x.dtype),
        scratch_shapes=[
            pltpu.VMEM((_NT, _TM, D), jnp.bfloat16),
            pltpu.VMEM((2, _TM, D), jnp.bfloat16),
            pltpu.VMEM((D, inter), jnp.bfloat16),
            pltpu.VMEM((inter, D), jnp.bfloat16),
            pltpu.VMEM((D, D), jnp.bfloat16),
            pltpu.SemaphoreType.DMA((_NT,)),
            pltpu.SemaphoreType.DMA((_NT,)),
            pltpu.SemaphoreType.DMA((3,)),
        ],
    )(x_flat, W1[0], W2[0])
    return out.reshape(Bq, S, D)


# speedup vs baseline: 1.1349x; 1.1349x over previous
import jax
import jax.numpy as jnp
from jax.experimental import pallas as pl
from jax.experimental.pallas import tpu as pltpu

_TM = 1024
_NT = 8


def _fused_kernel(x_hbm, w1_ref, w2_ref, o_hbm,
                  xbuf, obuf, wc_ref, in_sems, out_sems):
    for c in range(_NT // 2):
        pltpu.make_async_copy(
            x_hbm.at[pl.ds(c * 2 * _TM, 2 * _TM), :],
            xbuf.at[pl.ds(c * 2 * _TM, 2 * _TM), :], in_sems.at[c]
        ).start()

    wc_ref[...] = jax.lax.dot_general(
        w1_ref[...], w2_ref[...],
        dimension_numbers=(((1,), (1,)), ((), ())),
        preferred_element_type=jnp.float32).astype(jnp.bfloat16)

    for i in range(_NT):
        if i % 2 == 0:
            pltpu.make_async_copy(
                x_hbm.at[pl.ds(i * _TM, 2 * _TM), :],
                xbuf.at[pl.ds(i * _TM, 2 * _TM), :], in_sems.at[i // 2]
            ).wait()
        slot = i % 2
        if i >= 2:
            pltpu.make_async_copy(
                obuf.at[slot], o_hbm.at[pl.ds((i - 2) * _TM, _TM), :],
                out_sems.at[i - 2]
            ).wait()
        obuf[slot] = jnp.dot(
            xbuf[pl.ds(i * _TM, _TM), :], wc_ref[...],
            preferred_element_type=jnp.float32).astype(jnp.bfloat16)
        pltpu.make_async_copy(
            obuf.at[slot], o_hbm.at[pl.ds(i * _TM, _TM), :], out_sems.at[i]
        ).start()

    for i in range(_NT - 2, _NT):
        pltpu.make_async_copy(
            obuf.at[i % 2], o_hbm.at[pl.ds(i * _TM, _TM), :], out_sems.at[i]
        ).wait()


def kernel(x, gate_w, bias, W1, W2):
    Bq, S, D = x.shape
    x_flat = x.reshape(-1, D)
    T = x_flat.shape[0]
    inter = W1.shape[2]
    out = pl.pallas_call(
        _fused_kernel,
        grid=(1,),
        in_specs=[
            pl.BlockSpec(memory_space=pl.ANY),
            pl.BlockSpec((D, inter), lambda i: (0, 0)),
            pl.BlockSpec((inter, D), lambda i: (0, 0)),
        ],
        out_specs=pl.BlockSpec(memory_space=pl.ANY),
        out_shape=jax.ShapeDtypeStruct((T, D), x.dtype),
        scratch_shapes=[
            pltpu.VMEM((_NT * _TM, D), jnp.bfloat16),
            pltpu.VMEM((2, _TM, D), jnp.bfloat16),
            pltpu.VMEM((D, D), jnp.bfloat16),
            pltpu.SemaphoreType.DMA((_NT,)),
            pltpu.SemaphoreType.DMA((_NT,)),
        ],
    )(x_flat, W1[0], W2[0])
    return out.reshape(Bq, S, D)
